# Initial kernel scaffold; baseline (speedup 1.0000x reference)
import functools

import jax
import jax.numpy as jnp
from jax import lax
from jax.experimental import pallas as pl
from jax.experimental.pallas import tpu as pltpu
from jax.experimental.pallas import tpu_sc as plsc

_N = 10000
_E = 320000
_D = 128

_NC = 2
_NS = 16
_NW = _NC * _NS

_CHUNK = 128
_CHUNKS = 79
_EPAD = _NW * _CHUNKS * _CHUNK
_NPAD = 10240
_ROWCHUNKS = _NPAD // _CHUNK
_DUMMY = _N

_DEGW = 16


def _sc_agg_body(with_deg, *refs):
    if with_deg:
        (h_hbm, src_hbm, dst_hbm, out_hbm, deg_hbm,
         src_v, dst_v, rows_v, ones_v, zero16_v, acc_sh, deg_sh, sem) = refs
    else:
        (h_hbm, src_hbm, dst_hbm, out_hbm,
         src_v, dst_v, rows_v, acc_sh, sem) = refs

    c = lax.axis_index("c")
    s = lax.axis_index("s")
    wid = s * _NC + c

    pltpu.sync_copy(src_hbm.at[pl.ds(wid * _CHUNKS, _CHUNKS)], src_v)
    pltpu.sync_copy(dst_hbm.at[pl.ds(wid * _CHUNKS, _CHUNKS)], dst_v)

    zero16 = jnp.zeros((16,), jnp.float32)

    def _zrow(i, _):
        for q in range(_D // 16):
            rows_v[i, pl.ds(q * 16, 16)] = zero16
        return 0

    lax.fori_loop(0, _CHUNK, _zrow, 0)

    if with_deg:
        one16 = jnp.ones((16,), jnp.float32)

        def _onerow(i, _):
            ones_v[i, pl.ds(0, 16)] = one16
            zero16_v[i, pl.ds(0, 16)] = zero16
            return 0

        lax.fori_loop(0, _CHUNK, _onerow, 0)

    per_sub = _ROWCHUNKS // _NS
    for k in range(per_sub):
        chunk = s * per_sub + k
        pltpu.sync_copy(rows_v, acc_sh.at[pl.ds(chunk * _CHUNK, _CHUNK)])
        if with_deg:
            pltpu.sync_copy(zero16_v, deg_sh.at[pl.ds(chunk * _CHUNK, _CHUNK)])

    plsc.subcore_barrier()

    def _edge(j, _):
        pltpu.async_copy(h_hbm.at[src_v.at[j]], rows_v, sem).wait()
        pltpu.sync_copy(rows_v, acc_sh.at[dst_v.at[j]], add=True)
        if with_deg:
            pltpu.sync_copy(ones_v, deg_sh.at[dst_v.at[j]], add=True)
        return 0

    lax.fori_loop(0, _CHUNKS, _edge, 0)

    plsc.subcore_barrier()

    for k in range(per_sub):
        chunk = s * per_sub + k
        rows = pl.ds(chunk * _CHUNK, _CHUNK)
        pltpu.sync_copy(acc_sh.at[rows], out_hbm.at[c].at[rows])
        if with_deg:
            pltpu.sync_copy(deg_sh.at[rows], deg_hbm.at[c].at[rows])


def _make_sc_agg(with_deg):
    mesh = plsc.VectorSubcoreMesh(core_axis_name="c", subcore_axis_name="s")
    out_type = [jax.ShapeDtypeStruct((_NC, _NPAD, _D), jnp.float32)]
    scratch = [
        pltpu.VMEM((_CHUNKS, _CHUNK), jnp.int32),
        pltpu.VMEM((_CHUNKS, _CHUNK), jnp.int32),
        pltpu.VMEM((_CHUNK, _D), jnp.float32),
    ]
    if with_deg:
        out_type.append(jax.ShapeDtypeStruct((_NC, _NPAD, _DEGW), jnp.float32))
        scratch += [
            pltpu.VMEM((_CHUNK, _DEGW), jnp.float32),
            pltpu.VMEM((_CHUNK, _DEGW), jnp.float32),
        ]
    scratch.append(pltpu.VMEM_SHARED((_NPAD, _D), jnp.float32))
    if with_deg:
        scratch.append(pltpu.VMEM_SHARED((_NPAD, _DEGW), jnp.float32))
    scratch.append(pltpu.SemaphoreType.DMA)

    return pl.kernel(
        functools.partial(_sc_agg_body, with_deg),
        out_type=out_type,
        mesh=mesh,
        scratch_types=scratch,
    )


_BLK = 256


def _map_body(x_ref, w_ref, b_ref, o_ref):
    o_ref[...] = (
        jnp.dot(x_ref[...], w_ref[...], preferred_element_type=jnp.float32)
        + b_ref[...]
    )


def _tc_map(x, w, b):
    return pl.pallas_call(
        _map_body,
        grid=(_NPAD // _BLK,),
        in_specs=[
            pl.BlockSpec((_BLK, _D), lambda i: (i, 0)),
            pl.BlockSpec((_D, _D), lambda i: (0, 0)),
            pl.BlockSpec((1, _D), lambda i: (0, 0)),
        ],
        out_specs=pl.BlockSpec((_BLK, _D), lambda i: (i, 0)),
        out_shape=jax.ShapeDtypeStruct((_NPAD, _D), jnp.float32),
    )(x, w, b.reshape(1, _D))


def _layer_body(relu, agg_ref, deg_ref, h_ref, w_ref, b_ref, o_ref):
    deg = deg_ref[0][:, :1] + deg_ref[1][:, :1]
    hn = (agg_ref[0] + agg_ref[1] + h_ref[...]) / (deg + 1.0)
    out = (
        jnp.dot(hn, w_ref[...], preferred_element_type=jnp.float32)
        + b_ref[...]
    )
    if relu:
        out = jnp.maximum(out, 0.0)
    o_ref[...] = out


def _tc_layer(agg, deg, h, w, b, relu):
    return pl.pallas_call(
        functools.partial(_layer_body, relu),
        grid=(_NPAD // _BLK,),
        in_specs=[
            pl.BlockSpec((_NC, _BLK, _D), lambda i: (0, i, 0)),
            pl.BlockSpec((_NC, _BLK, _DEGW), lambda i: (0, i, 0)),
            pl.BlockSpec((_BLK, _D), lambda i: (i, 0)),
            pl.BlockSpec((_D, _D), lambda i: (0, 0)),
            pl.BlockSpec((1, _D), lambda i: (0, 0)),
        ],
        out_specs=pl.BlockSpec((_BLK, _D), lambda i: (i, 0)),
        out_shape=jax.ShapeDtypeStruct((_NPAD, _D), jnp.float32),
    )(agg, deg, h, w, b.reshape(1, _D))


def kernel(x, edge_index, W_map, b_map, W0, b0, W1, b1, W2, b2):
    src = edge_index[0]
    dst = edge_index[1]

    pad_e = _EPAD - _E
    src_p = jnp.concatenate([src, jnp.zeros((pad_e,), jnp.int32)])
    dst_p = jnp.concatenate([dst, jnp.full((pad_e,), _DUMMY, jnp.int32)])
    src_p = src_p.reshape(_EPAD // _CHUNK, _CHUNK)
    dst_p = dst_p.reshape(_EPAD // _CHUNK, _CHUNK)

    x_p = jnp.zeros((_NPAD, _D), jnp.float32).at[:_N].set(x)

    agg_deg = _make_sc_agg(True)
    agg_only = _make_sc_agg(False)

    h0 = _tc_map(x_p, W_map, b_map)
    agg, deg = agg_deg(h0, src_p, dst_p)
    h1 = _tc_layer(agg, deg, h0, W0, b0, True)
    (agg,) = agg_only(h1, src_p, dst_p)
    h2 = _tc_layer(agg, deg, h1, W1, b1, True)
    (agg,) = agg_only(h2, src_p, dst_p)
    h3 = _tc_layer(agg, deg, h2, W2, b2, False)
    return h3[:_N]


# speedup vs baseline: 2.9060x; 2.9060x over previous
import functools

import jax
import jax.numpy as jnp
from jax import lax
from jax.experimental import pallas as pl
from jax.experimental.pallas import tpu as pltpu
from jax.experimental.pallas import tpu_sc as plsc

_N = 10000
_E = 320000
_D = 128

_NC = 2
_NS = 16
_NW = _NC * _NS

_CHUNK = 128
_GRP = 8
_GRPS = 10
_CHUNKS = _GRP * _GRPS
_EPAD = _NW * _CHUNKS * _CHUNK
_NPAD = 10240
_ROWCHUNKS = _NPAD // _CHUNK
_DUMMY = _N

_DEGW = 16


def _sc_agg_body(h_hbm, src_hbm, dst_hbm, out_hbm,
                 src_v, dst_v, rows_v, acc_sh, sem):
    c = lax.axis_index("c")
    s = lax.axis_index("s")
    wid = s * _NC + c

    zero16 = jnp.zeros((16,), jnp.float32)

    def _zrow(i, _):
        for q in range(_D // 16):
            rows_v[i, pl.ds(q * 16, 16)] = zero16
        return 0

    lax.fori_loop(0, _CHUNK, _zrow, 0)

    per_sub = _ROWCHUNKS // _NS
    for k in range(per_sub):
        chunk = s * per_sub + k
        pltpu.sync_copy(rows_v, acc_sh.at[pl.ds(chunk * _CHUNK, _CHUNK)])

    plsc.subcore_barrier()

    def _group(g, _):
        base = wid * _CHUNKS + g * _GRP
        pltpu.sync_copy(src_hbm.at[pl.ds(base, _GRP)], src_v)
        pltpu.sync_copy(dst_hbm.at[pl.ds(base, _GRP)], dst_v)
        for j in range(_GRP):
            pltpu.async_copy(h_hbm.at[src_v.at[j]], rows_v, sem).wait()
            pltpu.sync_copy(rows_v, acc_sh.at[dst_v.at[j]], add=True)
        return 0

    lax.fori_loop(0, _GRPS, _group, 0)

    plsc.subcore_barrier()

    for k in range(per_sub):
        chunk = s * per_sub + k
        rows = pl.ds(chunk * _CHUNK, _CHUNK)
        pltpu.sync_copy(acc_sh.at[rows], out_hbm.at[c].at[rows])


_sc_agg = pl.kernel(
    _sc_agg_body,
    out_type=[jax.ShapeDtypeStruct((_NC, _NPAD, _D), jnp.float32)],
    mesh=plsc.VectorSubcoreMesh(core_axis_name="c", subcore_axis_name="s"),
    scratch_types=[
        pltpu.VMEM((_GRP, _CHUNK), jnp.int32),
        pltpu.VMEM((_GRP, _CHUNK), jnp.int32),
        pltpu.VMEM((_CHUNK, _D), jnp.float32),
        pltpu.VMEM_SHARED((_NPAD, _D), jnp.float32),
        pltpu.SemaphoreType.DMA,
    ],
)


def _sc_deg_body(dst_hbm, deg_hbm, dst_v, ones_v, deg_sh):
    c = lax.axis_index("c")
    s = lax.axis_index("s")
    wid = s * _NC + c

    zero16 = jnp.zeros((16,), jnp.float32)
    one16 = jnp.ones((16,), jnp.float32)

    def _fill(i, _):
        for q in range(_D // 16):
            ones_v[i, pl.ds(q * 16, 16)] = zero16
        return 0

    lax.fori_loop(0, _CHUNK, _fill, 0)

    per_sub = _ROWCHUNKS // _NS
    for k in range(per_sub):
        chunk = s * per_sub + k
        pltpu.sync_copy(ones_v, deg_sh.at[pl.ds(chunk * _CHUNK, _CHUNK)])

    def _fill1(i, _):
        for q in range(_D // 16):
            ones_v[i, pl.ds(q * 16, 16)] = one16
        return 0

    lax.fori_loop(0, _CHUNK, _fill1, 0)

    plsc.subcore_barrier()

    def _group(g, _):
        base = wid * _CHUNKS + g * _GRP
        pltpu.sync_copy(dst_hbm.at[pl.ds(base, _GRP)], dst_v)
        for j in range(_GRP):
            pltpu.sync_copy(ones_v, deg_sh.at[dst_v.at[j]], add=True)
        return 0

    lax.fori_loop(0, _GRPS, _group, 0)

    plsc.subcore_barrier()

    for k in range(per_sub):
        chunk = s * per_sub + k
        rows = pl.ds(chunk * _CHUNK, _CHUNK)
        pltpu.sync_copy(deg_sh.at[rows], deg_hbm.at[c].at[rows])


_sc_deg = pl.kernel(
    _sc_deg_body,
    out_type=[jax.ShapeDtypeStruct((_NC, _NPAD, _D), jnp.float32)],
    mesh=plsc.VectorSubcoreMesh(core_axis_name="c", subcore_axis_name="s"),
    scratch_types=[
        pltpu.VMEM((_GRP, _CHUNK), jnp.int32),
        pltpu.VMEM((_CHUNK, _D), jnp.float32),
        pltpu.VMEM_SHARED((_NPAD, _D), jnp.float32),
    ],
)


_BLK = 256


def _map_body(x_ref, w_ref, b_ref, o_ref):
    o_ref[...] = (
        jnp.dot(x_ref[...], w_ref[...], preferred_element_type=jnp.float32)
        + b_ref[...]
    )


def _tc_map(x, w, b):
    return pl.pallas_call(
        _map_body,
        grid=(_NPAD // _BLK,),
        in_specs=[
            pl.BlockSpec((_BLK, _D), lambda i: (i, 0)),
            pl.BlockSpec((_D, _D), lambda i: (0, 0)),
            pl.BlockSpec((1, _D), lambda i: (0, 0)),
        ],
        out_specs=pl.BlockSpec((_BLK, _D), lambda i: (i, 0)),
        out_shape=jax.ShapeDtypeStruct((_NPAD, _D), jnp.float32),
    )(x, w, b.reshape(1, _D))


def _layer_body(relu, agg_ref, deg_ref, h_ref, w_ref, b_ref, o_ref):
    deg = deg_ref[0][:, :1] + deg_ref[1][:, :1]
    hn = (agg_ref[0] + agg_ref[1] + h_ref[...]) / (deg + 1.0)
    out = (
        jnp.dot(hn, w_ref[...], preferred_element_type=jnp.float32)
        + b_ref[...]
    )
    if relu:
        out = jnp.maximum(out, 0.0)
    o_ref[...] = out


def _tc_layer(agg, deg, h, w, b, relu):
    return pl.pallas_call(
        functools.partial(_layer_body, relu),
        grid=(_NPAD // _BLK,),
        in_specs=[
            pl.BlockSpec((_NC, _BLK, _D), lambda i: (0, i, 0)),
            pl.BlockSpec((_NC, _BLK, _D), lambda i: (0, i, 0)),
            pl.BlockSpec((_BLK, _D), lambda i: (i, 0)),
            pl.BlockSpec((_D, _D), lambda i: (0, 0)),
            pl.BlockSpec((1, _D), lambda i: (0, 0)),
        ],
        out_specs=pl.BlockSpec((_BLK, _D), lambda i: (i, 0)),
        out_shape=jax.ShapeDtypeStruct((_NPAD, _D), jnp.float32),
    )(agg, deg, h, w, b.reshape(1, _D))


def kernel(x, edge_index, W_map, b_map, W0, b0, W1, b1, W2, b2):
    src = edge_index[0]
    dst = edge_index[1]

    pad_e = _EPAD - _E
    src_p = jnp.concatenate([src, jnp.zeros((pad_e,), jnp.int32)])
    dst_p = jnp.concatenate([dst, jnp.full((pad_e,), _DUMMY, jnp.int32)])
    src_p = src_p.reshape(_EPAD // _CHUNK, _CHUNK)
    dst_p = dst_p.reshape(_EPAD // _CHUNK, _CHUNK)

    x_p = jnp.zeros((_NPAD, _D), jnp.float32).at[:_N].set(x)

    (deg,) = _sc_deg(dst_p)
    h0 = _tc_map(x_p, W_map, b_map)
    (agg,) = _sc_agg(h0, src_p, dst_p)
    h1 = _tc_layer(agg, deg, h0, W0, b0, True)
    (agg,) = _sc_agg(h1, src_p, dst_p)
    h2 = _tc_layer(agg, deg, h1, W1, b1, True)
    (agg,) = _sc_agg(h2, src_p, dst_p)
    h3 = _tc_layer(agg, deg, h2, W2, b2, False)
    return h3[:_N]


# speedup vs baseline: 3.2712x; 1.1257x over previous
import functools

import jax
import jax.numpy as jnp
from jax import lax
from jax.experimental import pallas as pl
from jax.experimental.pallas import tpu as pltpu
from jax.experimental.pallas import tpu_sc as plsc

_N = 10000
_E = 320000
_D = 128

_NC = 2
_NS = 16
_NW = _NC * _NS

_CHUNK = 128
_GRP = 8
_GRPS = 10
_CHUNKS = _GRP * _GRPS
_EPAD = _NW * _CHUNKS * _CHUNK
_NPAD = 10240
_ROWCHUNKS = _NPAD // _CHUNK
_DUMMY = _N

_DEGW = 16


def _sc_agg_body(h_hbm, src_hbm, dst_hbm, out_hbm,
                 src_v, dst_v, rows_a, rows_b, acc_sh, sem_a, sem_b):
    c = lax.axis_index("c")
    s = lax.axis_index("s")
    wid = s * _NC + c
    base = wid * _CHUNKS

    zero16 = jnp.zeros((16,), jnp.float32)

    def _zrow(i, _):
        for q in range(_D // 16):
            rows_a[i, pl.ds(q * 16, 16)] = zero16
        return 0

    lax.fori_loop(0, _CHUNK, _zrow, 0)

    per_sub = _ROWCHUNKS // _NS
    for k in range(per_sub):
        chunk = s * per_sub + k
        pltpu.sync_copy(rows_a, acc_sh.at[pl.ds(chunk * _CHUNK, _CHUNK)])

    pltpu.sync_copy(src_hbm.at[pl.ds(base, _CHUNKS)], src_v)

    plsc.subcore_barrier()

    bufs = (rows_a, rows_b)
    sems = (sem_a, sem_b)

    pltpu.sync_copy(dst_hbm.at[pl.ds(base, _GRP)], dst_v)
    pltpu.async_copy(h_hbm.at[src_v.at[0]], rows_a, sem_a)

    def _group(g, _):
        for k in range(_GRP):
            cur, nxt = bufs[k % 2], bufs[1 - k % 2]
            csem, nsem = sems[k % 2], sems[1 - k % 2]
            nc = g * _GRP + k + 1
            if k < _GRP - 1:
                pltpu.async_copy(h_hbm.at[src_v.at[nc]], nxt, nsem)
            else:
                @pl.when(g < _GRPS - 1)
                def _():
                    pltpu.async_copy(h_hbm.at[src_v.at[nc]], nxt, nsem)
            pltpu.make_async_copy(h_hbm.at[src_v.at[0]], cur, csem).wait()
            pltpu.sync_copy(cur, acc_sh.at[dst_v.at[k]], add=True)

        @pl.when(g < _GRPS - 1)
        def _():
            pltpu.sync_copy(
                dst_hbm.at[pl.ds(base + (g + 1) * _GRP, _GRP)], dst_v)
        return 0

    lax.fori_loop(0, _GRPS, _group, 0)

    plsc.subcore_barrier()

    for k in range(per_sub):
        chunk = s * per_sub + k
        rows = pl.ds(chunk * _CHUNK, _CHUNK)
        pltpu.sync_copy(acc_sh.at[rows], out_hbm.at[c].at[rows])


_sc_agg = pl.kernel(
    _sc_agg_body,
    out_type=[jax.ShapeDtypeStruct((_NC, _NPAD, _D), jnp.float32)],
    mesh=plsc.VectorSubcoreMesh(core_axis_name="c", subcore_axis_name="s"),
    scratch_types=[
        pltpu.VMEM((_CHUNKS, _CHUNK), jnp.int32),
        pltpu.VMEM((_GRP, _CHUNK), jnp.int32),
        pltpu.VMEM((_CHUNK, _D), jnp.float32),
        pltpu.VMEM((_CHUNK, _D), jnp.float32),
        pltpu.VMEM_SHARED((_NPAD, _D), jnp.float32),
        pltpu.SemaphoreType.DMA,
        pltpu.SemaphoreType.DMA,
    ],
)


def _sc_deg_body(dst_hbm, deg_hbm, dst_v, ones_v, deg_sh):
    c = lax.axis_index("c")
    s = lax.axis_index("s")
    wid = s * _NC + c

    zero16 = jnp.zeros((16,), jnp.float32)
    one16 = jnp.ones((16,), jnp.float32)

    def _fill(i, _):
        for q in range(_D // 16):
            ones_v[i, pl.ds(q * 16, 16)] = zero16
        return 0

    lax.fori_loop(0, _CHUNK, _fill, 0)

    per_sub = _ROWCHUNKS // _NS
    for k in range(per_sub):
        chunk = s * per_sub + k
        pltpu.sync_copy(ones_v, deg_sh.at[pl.ds(chunk * _CHUNK, _CHUNK)])

    def _fill1(i, _):
        for q in range(_D // 16):
            ones_v[i, pl.ds(q * 16, 16)] = one16
        return 0

    lax.fori_loop(0, _CHUNK, _fill1, 0)

    plsc.subcore_barrier()

    def _group(g, _):
        base = wid * _CHUNKS + g * _GRP
        pltpu.sync_copy(dst_hbm.at[pl.ds(base, _GRP)], dst_v)
        for j in range(_GRP):
            pltpu.sync_copy(ones_v, deg_sh.at[dst_v.at[j]], add=True)
        return 0

    lax.fori_loop(0, _GRPS, _group, 0)

    plsc.subcore_barrier()

    for k in range(per_sub):
        chunk = s * per_sub + k
        rows = pl.ds(chunk * _CHUNK, _CHUNK)
        pltpu.sync_copy(deg_sh.at[rows], deg_hbm.at[c].at[rows])


_sc_deg = pl.kernel(
    _sc_deg_body,
    out_type=[jax.ShapeDtypeStruct((_NC, _NPAD, _D), jnp.float32)],
    mesh=plsc.VectorSubcoreMesh(core_axis_name="c", subcore_axis_name="s"),
    scratch_types=[
        pltpu.VMEM((_GRP, _CHUNK), jnp.int32),
        pltpu.VMEM((_CHUNK, _D), jnp.float32),
        pltpu.VMEM_SHARED((_NPAD, _D), jnp.float32),
    ],
)


_BLK = 256


def _map_body(x_ref, w_ref, b_ref, o_ref):
    o_ref[...] = (
        jnp.dot(x_ref[...], w_ref[...], preferred_element_type=jnp.float32)
        + b_ref[...]
    )


def _tc_map(x, w, b):
    return pl.pallas_call(
        _map_body,
        grid=(_NPAD // _BLK,),
        in_specs=[
            pl.BlockSpec((_BLK, _D), lambda i: (i, 0)),
            pl.BlockSpec((_D, _D), lambda i: (0, 0)),
            pl.BlockSpec((1, _D), lambda i: (0, 0)),
        ],
        out_specs=pl.BlockSpec((_BLK, _D), lambda i: (i, 0)),
        out_shape=jax.ShapeDtypeStruct((_NPAD, _D), jnp.float32),
    )(x, w, b.reshape(1, _D))


def _layer_body(relu, agg_ref, deg_ref, h_ref, w_ref, b_ref, o_ref):
    deg = deg_ref[0][:, :1] + deg_ref[1][:, :1]
    hn = (agg_ref[0] + agg_ref[1] + h_ref[...]) / (deg + 1.0)
    out = (
        jnp.dot(hn, w_ref[...], preferred_element_type=jnp.float32)
        + b_ref[...]
    )
    if relu:
        out = jnp.maximum(out, 0.0)
    o_ref[...] = out


def _tc_layer(agg, deg, h, w, b, relu):
    return pl.pallas_call(
        functools.partial(_layer_body, relu),
        grid=(_NPAD // _BLK,),
        in_specs=[
            pl.BlockSpec((_NC, _BLK, _D), lambda i: (0, i, 0)),
            pl.BlockSpec((_NC, _BLK, _D), lambda i: (0, i, 0)),
            pl.BlockSpec((_BLK, _D), lambda i: (i, 0)),
            pl.BlockSpec((_D, _D), lambda i: (0, 0)),
            pl.BlockSpec((1, _D), lambda i: (0, 0)),
        ],
        out_specs=pl.BlockSpec((_BLK, _D), lambda i: (i, 0)),
        out_shape=jax.ShapeDtypeStruct((_NPAD, _D), jnp.float32),
    )(agg, deg, h, w, b.reshape(1, _D))


def kernel(x, edge_index, W_map, b_map, W0, b0, W1, b1, W2, b2):
    src = edge_index[0]
    dst = edge_index[1]

    pad_e = _EPAD - _E
    src_p = jnp.concatenate([src, jnp.zeros((pad_e,), jnp.int32)])
    dst_p = jnp.concatenate([dst, jnp.full((pad_e,), _DUMMY, jnp.int32)])
    src_p = src_p.reshape(_EPAD // _CHUNK, _CHUNK)
    dst_p = dst_p.reshape(_EPAD // _CHUNK, _CHUNK)

    x_p = jnp.zeros((_NPAD, _D), jnp.float32).at[:_N].set(x)

    (deg,) = _sc_deg(dst_p)
    h0 = _tc_map(x_p, W_map, b_map)
    (agg,) = _sc_agg(h0, src_p, dst_p)
    h1 = _tc_layer(agg, deg, h0, W0, b0, True)
    (agg,) = _sc_agg(h1, src_p, dst_p)
    h2 = _tc_layer(agg, deg, h1, W1, b1, True)
    (agg,) = _sc_agg(h2, src_p, dst_p)
    h3 = _tc_layer(agg, deg, h2, W2, b2, False)
    return h3[:_N]


# speedup vs baseline: 4.2440x; 1.2974x over previous
import functools

import jax
import jax.numpy as jnp
from jax import lax
from jax.experimental import pallas as pl
from jax.experimental.pallas import tpu as pltpu
from jax.experimental.pallas import tpu_sc as plsc

_N = 10000
_E = 320000
_D = 128

_NC = 2
_NS = 16
_NW = _NC * _NS

_CHUNK = 128
_GRP = 8
_GRPS = 10
_CHUNKS = _GRP * _GRPS
_EPAD = _NW * _CHUNKS * _CHUNK
_NPAD = 10240
_ROWCHUNKS = _NPAD // _CHUNK
_DUMMY = _N

_CH_C0 = 120
_CH_C1 = 40
_CH_PAIR = _CH_C0 + _CH_C1
_ACC_CHUNKS = 79
_NDUMMY = _ACC_CHUNKS * _CHUNK - _N


def _sc_agg_body(h_hbm, src_hbm, dst_hbm, out_hbm,
                 src_v, dst_v, rows_a, rows_b, acc_sh, sem_a, sem_b):
    c = lax.axis_index("c")
    s = lax.axis_index("s")
    base = s * _CH_PAIR + c * _CH_C0

    zero16 = jnp.zeros((16,), jnp.float32)

    def _zrow(i, _):
        for q in range(_D // 16):
            rows_a[i, pl.ds(q * 16, 16)] = zero16
        return 0

    lax.fori_loop(0, _CHUNK, _zrow, 0)

    per_sub = 5
    for k in range(per_sub):
        chunk = s * per_sub + k

        @pl.when(chunk < _ACC_CHUNKS)
        def _():
            pltpu.sync_copy(rows_a, acc_sh.at[pl.ds(chunk * _CHUNK, _CHUNK)])

    @pl.when(c == 0)
    def _():
        pltpu.sync_copy(src_hbm.at[pl.ds(base, _CH_C0)],
                        src_v.at[pl.ds(0, _CH_C0)])

    @pl.when(c == 1)
    def _():
        pltpu.sync_copy(src_hbm.at[pl.ds(base, _CH_C1)],
                        src_v.at[pl.ds(0, _CH_C1)])

    plsc.subcore_barrier()

    bufs = (rows_a, rows_b)
    sems = (sem_a, sem_b)

    half = _CHUNK // 2

    def _gather(nc, buf, sem):
        pltpu.async_copy(h_hbm.at[src_v.at[nc, pl.ds(0, half)]],
                         buf.at[pl.ds(0, half)], sem)
        pltpu.async_copy(h_hbm.at[src_v.at[nc, pl.ds(half, half)]],
                         buf.at[pl.ds(half, half)], sem)

    def _edge_loop(ngrp):
        pltpu.sync_copy(dst_hbm.at[pl.ds(base, _GRP)], dst_v)
        _gather(0, rows_a, sem_a)

        def _group(g, _):
            for k in range(_GRP):
                cur, nxt = bufs[k % 2], bufs[1 - k % 2]
                csem, nsem = sems[k % 2], sems[1 - k % 2]
                nc = g * _GRP + k + 1
                if k < _GRP - 1:
                    _gather(nc, nxt, nsem)
                else:
                    @pl.when(g < ngrp - 1)
                    def _():
                        _gather(nc, nxt, nsem)
                pltpu.make_async_copy(h_hbm.at[src_v.at[0]], cur, csem).wait()
                pltpu.sync_copy(cur, acc_sh.at[dst_v.at[k]], add=True)

            @pl.when(g < ngrp - 1)
            def _():
                pltpu.sync_copy(
                    dst_hbm.at[pl.ds(base + (g + 1) * _GRP, _GRP)], dst_v)
            return 0

        lax.fori_loop(0, ngrp, _group, 0)

    @pl.when(c == 0)
    def _():
        _edge_loop(_CH_C0 // _GRP)

    @pl.when(c == 1)
    def _():
        _edge_loop(_CH_C1 // _GRP)

    plsc.subcore_barrier()

    for k in range(per_sub):
        chunk = s * per_sub + k

        @pl.when(chunk < _ACC_CHUNKS)
        def _():
            rows = pl.ds(chunk * _CHUNK, _CHUNK)
            pltpu.sync_copy(acc_sh.at[rows], out_hbm.at[c].at[rows])


_sc_agg = pl.kernel(
    _sc_agg_body,
    out_type=[jax.ShapeDtypeStruct((_NC, _NPAD, _D), jnp.float32)],
    mesh=plsc.VectorSubcoreMesh(core_axis_name="c", subcore_axis_name="s"),
    scratch_types=[
        pltpu.VMEM((_CH_C0, _CHUNK), jnp.int32),
        pltpu.VMEM((_GRP, _CHUNK), jnp.int32),
        pltpu.VMEM((_CHUNK, _D), jnp.float32),
        pltpu.VMEM((_CHUNK, _D), jnp.float32),
        pltpu.VMEM_SHARED((_ACC_CHUNKS * _CHUNK, _D), jnp.float32),
        pltpu.SemaphoreType.DMA,
        pltpu.SemaphoreType.DMA,
    ],
)


def _sc_deg_body(dst_hbm, deg_hbm, dst_v, ones_v, deg_sh):
    c = lax.axis_index("c")
    s = lax.axis_index("s")
    wid = s * _NC + c

    zero16 = jnp.zeros((16,), jnp.float32)
    one16 = jnp.ones((16,), jnp.float32)

    def _fill(i, _):
        for q in range(_D // 16):
            ones_v[i, pl.ds(q * 16, 16)] = zero16
        return 0

    lax.fori_loop(0, _CHUNK, _fill, 0)

    per_sub = _ROWCHUNKS // _NS
    for k in range(per_sub):
        chunk = s * per_sub + k
        pltpu.sync_copy(ones_v, deg_sh.at[pl.ds(chunk * _CHUNK, _CHUNK)])

    def _fill1(i, _):
        for q in range(_D // 16):
            ones_v[i, pl.ds(q * 16, 16)] = one16
        return 0

    lax.fori_loop(0, _CHUNK, _fill1, 0)

    plsc.subcore_barrier()

    def _group(g, _):
        base = wid * _CHUNKS + g * _GRP
        pltpu.sync_copy(dst_hbm.at[pl.ds(base, _GRP)], dst_v)
        for j in range(_GRP):
            pltpu.sync_copy(ones_v, deg_sh.at[dst_v.at[j]], add=True)
        return 0

    lax.fori_loop(0, _GRPS, _group, 0)

    plsc.subcore_barrier()

    for k in range(per_sub):
        chunk = s * per_sub + k
        rows = pl.ds(chunk * _CHUNK, _CHUNK)
        pltpu.sync_copy(deg_sh.at[rows], deg_hbm.at[c].at[rows])


_sc_deg = pl.kernel(
    _sc_deg_body,
    out_type=[jax.ShapeDtypeStruct((_NC, _NPAD, _D), jnp.float32)],
    mesh=plsc.VectorSubcoreMesh(core_axis_name="c", subcore_axis_name="s"),
    scratch_types=[
        pltpu.VMEM((_GRP, _CHUNK), jnp.int32),
        pltpu.VMEM((_CHUNK, _D), jnp.float32),
        pltpu.VMEM_SHARED((_NPAD, _D), jnp.float32),
    ],
)


_BLK = 1024


def _map_body(x_ref, w_ref, b_ref, o_ref):
    o_ref[...] = (
        jnp.dot(x_ref[...], w_ref[...], preferred_element_type=jnp.float32)
        + b_ref[...]
    )


def _tc_map(x, w, b):
    return pl.pallas_call(
        _map_body,
        grid=(_NPAD // _BLK,),
        in_specs=[
            pl.BlockSpec((_BLK, _D), lambda i: (i, 0)),
            pl.BlockSpec((_D, _D), lambda i: (0, 0)),
            pl.BlockSpec((1, _D), lambda i: (0, 0)),
        ],
        out_specs=pl.BlockSpec((_BLK, _D), lambda i: (i, 0)),
        out_shape=jax.ShapeDtypeStruct((_NPAD, _D), jnp.float32),
    )(x, w, b.reshape(1, _D))


def _scale_body(deg_ref, s_ref):
    deg = deg_ref[0][:, :1] + deg_ref[1][:, :1]
    s_ref[...] = 1.0 / (deg + 1.0)


def _tc_scale(deg):
    return pl.pallas_call(
        _scale_body,
        grid=(_NPAD // _BLK,),
        in_specs=[pl.BlockSpec((_NC, _BLK, _D), lambda i: (0, i, 0))],
        out_specs=pl.BlockSpec((_BLK, 1), lambda i: (i, 0)),
        out_shape=jax.ShapeDtypeStruct((_NPAD, 1), jnp.float32),
    )(deg)


def _layer_body(relu, agg_ref, s_ref, h_ref, w_ref, b_ref, o_ref):
    hn = (agg_ref[0] + agg_ref[1] + h_ref[...]) * s_ref[...]
    out = (
        jnp.dot(hn, w_ref[...], preferred_element_type=jnp.float32)
        + b_ref[...]
    )
    if relu:
        out = jnp.maximum(out, 0.0)
    o_ref[...] = out


def _tc_layer(agg, scale, h, w, b, relu):
    return pl.pallas_call(
        functools.partial(_layer_body, relu),
        grid=(_NPAD // _BLK,),
        in_specs=[
            pl.BlockSpec((_NC, _BLK, _D), lambda i: (0, i, 0)),
            pl.BlockSpec((_BLK, 1), lambda i: (i, 0)),
            pl.BlockSpec((_BLK, _D), lambda i: (i, 0)),
            pl.BlockSpec((_D, _D), lambda i: (0, 0)),
            pl.BlockSpec((1, _D), lambda i: (0, 0)),
        ],
        out_specs=pl.BlockSpec((_BLK, _D), lambda i: (i, 0)),
        out_shape=jax.ShapeDtypeStruct((_NPAD, _D), jnp.float32),
    )(agg, scale, h, w, b.reshape(1, _D))


def kernel(x, edge_index, W_map, b_map, W0, b0, W1, b1, W2, b2):
    src = edge_index[0]
    dst = edge_index[1]

    pad_e = _EPAD - _E
    pad_dst = _DUMMY + jnp.arange(pad_e, dtype=jnp.int32) % _NDUMMY
    src_p = jnp.concatenate([src, jnp.zeros((pad_e,), jnp.int32)])
    dst_p = jnp.concatenate([dst, pad_dst])
    src_p = src_p.reshape(_EPAD // _CHUNK, _CHUNK)
    dst_p = dst_p.reshape(_EPAD // _CHUNK, _CHUNK)

    x_p = jnp.zeros((_NPAD, _D), jnp.float32).at[:_N].set(x)

    (deg,) = _sc_deg(dst_p)
    scale = _tc_scale(deg)
    h0 = _tc_map(x_p, W_map, b_map)
    (agg,) = _sc_agg(h0, src_p, dst_p)
    h1 = _tc_layer(agg, scale, h0, W0, b0, True)
    (agg,) = _sc_agg(h1, src_p, dst_p)
    h2 = _tc_layer(agg, scale, h1, W1, b1, True)
    (agg,) = _sc_agg(h2, src_p, dst_p)
    h3 = _tc_layer(agg, scale, h2, W2, b2, False)
    return h3[:_N]
